# baseline (speedup 1.0000x reference)
import jax
import jax.numpy as jnp
from jax.experimental import pallas as pl

_F_IN = 128
_F_OUT = 32
_NSPLIT = 2


def _fused_body(x0_ref, x1_ref, wxz_ref, bz_ref, wxh_ref, bh_ref, wl_ref,
                bl_ref, o_ref):
    wz = wxz_ref[0, 0, :_F_IN, :] + wxz_ref[1, 0, :_F_IN, :]
    wh = wxh_ref[0, 0, :_F_IN, :] + wxh_ref[1, 0, :_F_IN, :]
    half = x0_ref.shape[0]
    for j, x_ref in enumerate((x0_ref, x1_ref)):
        xb = x_ref[...]
        z = jax.nn.sigmoid(
            jnp.dot(xb, wz, preferred_element_type=jnp.float32) + bz_ref[...])
        h_tilde = jnp.tanh(
            jnp.dot(xb, wh, preferred_element_type=jnp.float32) + bh_ref[...])
        h = jnp.maximum((1.0 - z) * h_tilde, 0.0)
        o_ref[pl.ds(j * half, half), :] = jnp.dot(
            h, wl_ref[...], preferred_element_type=jnp.float32) + bl_ref[...]


def kernel(x, edge_index, W_xz, b_z, W_xr, b_r, W_xh, b_h, W_lin, b_lin):
    n = x.shape[0]
    half = n // _NSPLIT
    d_cat = W_xz.shape[2]
    whole = lambda *shape: pl.BlockSpec(shape, lambda i: (0,) * len(shape))
    return pl.pallas_call(
        _fused_body,
        grid=(1,),
        in_specs=[
            pl.BlockSpec((half, _F_IN), lambda i: (0, 0)),
            pl.BlockSpec((half, _F_IN), lambda i: (1, 0)),
            whole(2, 1, d_cat, _F_OUT),
            whole(_F_OUT),
            whole(2, 1, d_cat, _F_OUT),
            whole(_F_OUT),
            whole(_F_OUT, 1),
            whole(1),
        ],
        out_specs=pl.BlockSpec((n, 1), lambda i: (0, 0)),
        out_shape=jax.ShapeDtypeStruct((n, 1), jnp.float32),
    )(x, x, W_xz, b_z, W_xh, b_h, W_lin, b_lin)


# speedup vs baseline: 1.1654x; 1.1654x over previous
import jax
import jax.numpy as jnp
from jax.experimental import pallas as pl

_F_IN = 128
_F_OUT = 32
_BLOCK = 10000


def _fused_body(x_ref, wxz_ref, bz_ref, wxh_ref, bh_ref, wl_ref, bl_ref,
                o_ref):
    wz = wxz_ref[0, 0, :_F_IN, :] + wxz_ref[1, 0, :_F_IN, :]
    wh = wxh_ref[0, 0, :_F_IN, :] + wxh_ref[1, 0, :_F_IN, :]
    w = jnp.concatenate([0.5 * wz, wh], axis=1)
    b = jnp.concatenate([0.5 * bz_ref[...], bh_ref[...]])
    xb = x_ref[...]
    t = jnp.tanh(jnp.dot(xb, w, preferred_element_type=jnp.float32) + b)
    g = (1.0 - t[:, :_F_OUT]) * t[:, _F_OUT:]
    h = jnp.maximum(g, 0.0)
    o_ref[...] = jnp.dot(h, 0.5 * wl_ref[...],
                         preferred_element_type=jnp.float32) + bl_ref[...]


def kernel(x, edge_index, W_xz, b_z, W_xr, b_r, W_xh, b_h, W_lin, b_lin):
    n = x.shape[0]
    d_cat = W_xz.shape[2]
    grid = (n // _BLOCK,)
    whole = lambda *shape: pl.BlockSpec(shape, lambda i: (0,) * len(shape))
    return pl.pallas_call(
        _fused_body,
        grid=grid,
        in_specs=[
            pl.BlockSpec((_BLOCK, _F_IN), lambda i: (i, 0)),
            whole(2, 1, d_cat, _F_OUT),
            whole(_F_OUT),
            whole(2, 1, d_cat, _F_OUT),
            whole(_F_OUT),
            whole(_F_OUT, 1),
            whole(1),
        ],
        out_specs=pl.BlockSpec((_BLOCK, 1), lambda i: (i, 0)),
        out_shape=jax.ShapeDtypeStruct((n, 1), jnp.float32),
    )(x, W_xz, b_z, W_xh, b_h, W_lin, b_lin)
